# Initial kernel scaffold; baseline (speedup 1.0000x reference)
import jax
import jax.numpy as jnp
from jax.experimental import pallas as pl


def kernel(x, scale):
    raise NotImplementedError("write your pallas kernel here")


# speedup vs baseline: 81.8182x; 81.8182x over previous
import functools

import jax
import jax.numpy as jnp
from jax import lax
from jax.experimental import pallas as pl
from jax.experimental.pallas import tpu as pltpu
from jax.experimental.pallas import tpu_sc as plsc

_GAMMA = 0.999
_P = 0.9995
_N = 2 * 4096 * 2048
_RANK = int(_N * _P) - 1
_NEED = _N - _RANK

_L = 16
_NC, _NS = 2, 16
_NW = _NC * _NS
_PER_W = _N // _NW
_CHUNK = 65536
_NCHUNK = _PER_W // _CHUNK

_mesh = plsc.VectorSubcoreMesh(core_axis_name="c", subcore_axis_name="s")


def _wid():
    return lax.axis_index("s") * _NC + lax.axis_index("c")


@functools.partial(
    pl.kernel,
    out_type=(
        jax.ShapeDtypeStruct((_N,), jnp.float32),
        jax.ShapeDtypeStruct((_NW, _L), jnp.int32),
    ),
    mesh=_mesh,
    scratch_types=[
        pltpu.VMEM((_CHUNK,), jnp.float32),
        pltpu.VMEM((_L,), jnp.int32),
    ],
)
def _clamp_count(x_hbm, out_hbm, cnt_hbm, buf, cnt_v):
    base = _wid() * _PER_W
    gamma = jnp.float32(_GAMMA)

    def chunk_body(k, acc):
        off = base + k * _CHUNK
        pltpu.sync_copy(x_hbm.at[pl.ds(off, _CHUNK)], buf)

        def body(i, a):
            v = buf[pl.ds(i * _L, _L)]
            lt = v < gamma
            buf[pl.ds(i * _L, _L)] = jnp.where(lt, v, gamma)
            return a + jnp.where(lt, 0, 1).astype(jnp.int32)

        acc = lax.fori_loop(0, _CHUNK // _L, body, acc)
        pltpu.sync_copy(buf, out_hbm.at[pl.ds(off, _CHUNK)])
        return acc

    acc = lax.fori_loop(0, _NCHUNK, chunk_body, jnp.zeros((_L,), jnp.int32))
    cnt_v[...] = acc
    pltpu.sync_copy(cnt_v, cnt_hbm.at[_wid()])


def _make_hist_kernel(level):
    shift = 8 * level

    @functools.partial(
        pl.kernel,
        out_type=jax.ShapeDtypeStruct((_NW, 256 * _L), jnp.int32),
        mesh=_mesh,
        scratch_types=[
            pltpu.VMEM((_CHUNK,), jnp.float32),
            pltpu.VMEM((256 * _L,), jnp.int32),
            pltpu.VMEM((_L,), jnp.uint32),
        ],
        compiler_params=pltpu.CompilerParams(needs_layout_passes=False),
    )
    def hist_kernel(x_hbm, pref_hbm, hist_hbm, buf, hist_v, pref_v):
        base = _wid() * _PER_W
        gamma = jnp.float32(_GAMMA)
        lanes = jnp.arange(_L, dtype=jnp.int32)
        ones = jnp.ones((_L,), jnp.int32)
        zeros16 = jnp.zeros((_L,), jnp.int32)

        def zero_body(i, _):
            hist_v[pl.ds(i * _L, _L)] = zeros16
            return 0

        lax.fori_loop(0, 256, zero_body, 0)
        pltpu.sync_copy(pref_hbm, pref_v)
        pref = pref_v[...]

        def chunk_body(k, _):
            off = base + k * _CHUNK
            pltpu.sync_copy(x_hbm.at[pl.ds(off, _CHUNK)], buf)

            def body(i, __):
                v = buf[pl.ds(i * _L, _L)]
                u = lax.bitcast_convert_type(v, jnp.uint32)
                xm = jnp.where(
                    u >= jnp.uint32(0x80000000),
                    jnp.uint32(0xFFFFFFFF),
                    jnp.uint32(0x80000000),
                )
                key = u ^ xm
                mask = v < gamma
                if level < 3:
                    mask = mask & ((key >> jnp.uint32(shift + 8)) == pref)
                b = ((key >> jnp.uint32(shift)) & jnp.uint32(0xFF)).astype(jnp.int32)
                idx = b * _L + lanes
                plsc.addupdate_scatter(hist_v, [idx], ones, mask=mask)
                return 0

            lax.fori_loop(0, _CHUNK // _L, body, 0)
            return 0

        lax.fori_loop(0, _NCHUNK, chunk_body, 0)
        pltpu.sync_copy(hist_v, hist_hbm.at[_wid()])

    return hist_kernel


_hist_kernels = {lvl: _make_hist_kernel(lvl) for lvl in (3, 2, 1, 0)}


def kernel(x, scale):
    flat = x.reshape(-1)
    out_flat, cnts = _clamp_count(flat)
    ties = cnts.sum()

    def easy():
        return jnp.float32(_GAMMA)

    def hard():
        r = jnp.int32(_RANK)
        pref = jnp.uint32(0)
        for level in (3, 2, 1, 0):
            h = _hist_kernels[level](flat, jnp.full((_L,), pref, jnp.uint32))
            hh = h.reshape(_NW, 256, _L).sum(axis=(0, 2))
            cum = jnp.cumsum(hh)
            b = jnp.argmax(cum > r).astype(jnp.int32)
            r = r - (cum[b] - hh[b])
            pref = (pref << jnp.uint32(8)) | b.astype(jnp.uint32)
        u = jnp.where(
            pref >= jnp.uint32(0x80000000),
            pref ^ jnp.uint32(0x80000000),
            ~pref,
        )
        return lax.bitcast_convert_type(u, jnp.float32)

    val = lax.cond(ties >= _NEED, easy, hard)
    new_scale = jnp.maximum(val, scale)
    return out_flat.reshape(x.shape), new_scale
